# Initial kernel scaffold; baseline (speedup 1.0000x reference)
import functools

import jax
import jax.numpy as jnp
from jax import lax
from jax.experimental import pallas as pl
from jax.experimental.pallas import tpu as pltpu
from jax.experimental.pallas import tpu_sc as plsc

N = 10000
D = 128
E = 320000

NC = 2
NS = 16
NW = NC * NS

CHUNK = 128
CPT = 79
EPT = CPT * CHUNK
EP = EPT * NW
NPAD = 10240
RPT = NPAD // NS
BN = 512

_sc_mesh = plsc.VectorSubcoreMesh(core_axis_name="c", subcore_axis_name="s")


@functools.partial(
    pl.kernel,
    out_type=[
        jax.ShapeDtypeStruct((NC, NPAD), jnp.float32),
        jax.ShapeDtypeStruct((NC, NPAD), jnp.float32),
    ],
    mesh=_sc_mesh,
    scratch_types=[
        pltpu.VMEM((CPT, CHUNK), jnp.int32),
        pltpu.VMEM((CPT, CHUNK), jnp.int32),
        pltpu.VMEM((CHUNK,), jnp.float32),
        pltpu.VMEM((RPT,), jnp.float32),
        pltpu.VMEM_SHARED((NPAD,), jnp.float32),
        pltpu.VMEM_SHARED((NPAD,), jnp.float32),
    ],
)
def _count_kernel(srcm, dstm, out_s, out_d, src_v, dst_v, ones_v, drain_v,
                  acc_s, acc_d):
    c = lax.axis_index("c")
    s = lax.axis_index("s")
    w = s * NC + c

    def _zero(i, _):
        drain_v[pl.ds(i * 16, 16)] = jnp.zeros((16,), jnp.float32)
        return ()
    lax.fori_loop(0, RPT // 16, _zero, ())
    for j in range(CHUNK // 16):
        ones_v[pl.ds(j * 16, 16)] = jnp.ones((16,), jnp.float32)
    pltpu.sync_copy(drain_v, acc_s.at[pl.ds(s * RPT, RPT)])
    pltpu.sync_copy(drain_v, acc_d.at[pl.ds(s * RPT, RPT)])
    plsc.subcore_barrier()

    pltpu.sync_copy(srcm.at[pl.ds(w * CPT, CPT)], src_v)
    pltpu.sync_copy(dstm.at[pl.ds(w * CPT, CPT)], dst_v)

    def _body(g, _):
        pltpu.sync_copy(ones_v, acc_s.at[src_v.at[g]], add=True)
        pltpu.sync_copy(ones_v, acc_d.at[dst_v.at[g]], add=True)
        return ()
    lax.fori_loop(0, CPT, _body, ())

    plsc.subcore_barrier()
    pltpu.sync_copy(acc_s.at[pl.ds(s * RPT, RPT)], drain_v)
    pltpu.sync_copy(drain_v, out_s.at[c, pl.ds(s * RPT, RPT)])
    pltpu.sync_copy(acc_d.at[pl.ds(s * RPT, RPT)], drain_v)
    pltpu.sync_copy(drain_v, out_d.at[c, pl.ds(s * RPT, RPT)])


@functools.partial(
    pl.kernel,
    out_type=jax.ShapeDtypeStruct((NC, NPAD, D), jnp.float32),
    mesh=_sc_mesh,
    scratch_types=[
        pltpu.VMEM((CPT, CHUNK), jnp.int32),
        pltpu.VMEM((CPT, CHUNK), jnp.int32),
        pltpu.VMEM((CHUNK, D), jnp.float32),
        pltpu.VMEM_SHARED((NPAD, D), jnp.float32),
        pltpu.SemaphoreType.DMA,
    ],
)
def _scatter_kernel(xn, srcm, dstm, out, src_v, dst_v, rows_v, acc, sem):
    c = lax.axis_index("c")
    s = lax.axis_index("s")
    w = s * NC + c

    def _zrow(i, _):
        for j in range(D // 16):
            rows_v[i, pl.ds(j * 16, 16)] = jnp.zeros((16,), jnp.float32)
        return ()
    lax.fori_loop(0, CHUNK, _zrow, ())
    for k in range(RPT // CHUNK):
        pltpu.sync_copy(rows_v, acc.at[pl.ds(s * RPT + k * CHUNK, CHUNK)])
    plsc.subcore_barrier()

    pltpu.sync_copy(srcm.at[pl.ds(w * CPT, CPT)], src_v)
    pltpu.sync_copy(dstm.at[pl.ds(w * CPT, CPT)], dst_v)

    def _body(g, _):
        pltpu.async_copy(xn.at[src_v.at[g]], rows_v, sem).wait()
        pltpu.sync_copy(rows_v, acc.at[dst_v.at[g]], add=True)
        return ()
    lax.fori_loop(0, CPT, _body, ())

    plsc.subcore_barrier()
    for k in range(RPT // CHUNK):
        r0 = s * RPT + k * CHUNK
        pltpu.sync_copy(acc.at[pl.ds(r0, CHUNK)], rows_v)
        pltpu.sync_copy(rows_v, out.at[c, pl.ds(r0, CHUNK)])


def _norm_body(x_ref, cs_ref, xn_ref):
    deg_in = cs_ref[0] + cs_ref[1] + 1.0
    xn_ref[...] = x_ref[...] * lax.rsqrt(deg_in)[:, None]


_norm_call = pl.pallas_call(
    _norm_body,
    grid=(NPAD // BN,),
    in_specs=[
        pl.BlockSpec((BN, D), lambda i: (i, 0)),
        pl.BlockSpec((NC, BN), lambda i: (0, i)),
    ],
    out_specs=pl.BlockSpec((BN, D), lambda i: (i, 0)),
    out_shape=jax.ShapeDtypeStruct((NPAD, D), jnp.float32),
)


def _layer_body(p_ref, xn_ref, cs_ref, cd_ref, w_ref, b_ref,
                h_ref, hn_ref, g_ref):
    i = pl.program_id(0)
    deg_out = cd_ref[0] + cd_ref[1] + 1.0
    deg_in = cs_ref[0] + cs_ref[1] + 1.0
    upd = (p_ref[0] + p_ref[1] + xn_ref[...]) * lax.rsqrt(deg_out)[:, None]
    h = jnp.dot(upd, w_ref[...], preferred_element_type=jnp.float32)
    h = jnp.maximum(h + b_ref[...], 0.0)
    h_ref[...] = h
    hn_ref[...] = h * lax.rsqrt(deg_in)[:, None]
    rows = jax.lax.broadcasted_iota(jnp.int32, (BN, 1), 0) + i * BN
    bsum = jnp.sum(jnp.where(rows < N, h, 0.0), axis=0, keepdims=True)

    @pl.when(i == 0)
    def _():
        g_ref[...] = bsum

    @pl.when(i > 0)
    def _():
        g_ref[...] = g_ref[...] + bsum


_layer_call = pl.pallas_call(
    _layer_body,
    grid=(NPAD // BN,),
    in_specs=[
        pl.BlockSpec((NC, BN, D), lambda i: (0, i, 0)),
        pl.BlockSpec((BN, D), lambda i: (i, 0)),
        pl.BlockSpec((NC, BN), lambda i: (0, i)),
        pl.BlockSpec((NC, BN), lambda i: (0, i)),
        pl.BlockSpec((D, D), lambda i: (0, 0)),
        pl.BlockSpec((1, D), lambda i: (0, 0)),
    ],
    out_specs=[
        pl.BlockSpec((BN, D), lambda i: (i, 0)),
        pl.BlockSpec((BN, D), lambda i: (i, 0)),
        pl.BlockSpec((1, D), lambda i: (0, 0)),
    ],
    out_shape=[
        jax.ShapeDtypeStruct((NPAD, D), jnp.float32),
        jax.ShapeDtypeStruct((NPAD, D), jnp.float32),
        jax.ShapeDtypeStruct((1, D), jnp.float32),
    ],
)


def kernel(x, edge_index, W1, b1, W2, b2):
    src = edge_index[0]
    dst = edge_index[1]
    pad = jnp.full((EP - E,), N, jnp.int32)
    srcm = jnp.concatenate([src, pad]).reshape(EP // CHUNK, CHUNK)
    dstm = jnp.concatenate([dst, pad]).reshape(EP // CHUNK, CHUNK)
    xpad = jnp.zeros((NPAD, D), jnp.float32).at[:N].set(x)

    cs, cd = _count_kernel(srcm, dstm)
    xn = _norm_call(xpad, cs)
    p1 = _scatter_kernel(xn, srcm, dstm)
    _, hn1, _ = _layer_call(p1, xn, cs, cd, W1, b1.reshape(1, D))
    p2 = _scatter_kernel(hn1, srcm, dstm)
    h2, _, gsum = _layer_call(p2, hn1, cs, cd, W2, b2.reshape(1, D))
    return (gsum, h2[:N])


# speedup vs baseline: 6.6134x; 6.6134x over previous
import functools

import jax
import jax.numpy as jnp
from jax import lax
from jax.experimental import pallas as pl
from jax.experimental.pallas import tpu as pltpu
from jax.experimental.pallas import tpu_sc as plsc

N = 10000
D = 128
E = 320000

NC = 2
NS = 16
NW = NC * NS

CHUNK = 128
CPT = 80
EPT = CPT * CHUNK
EP = EPT * NW
NPAD = 10240
RPT = NPAD // NS
BN = 512

def _count_body(srcm, dstm, out_s, out_d, src_v, dst_v, ones_v, drain_v,
                acc_s, acc_d):
    c = lax.axis_index("c")
    s = lax.axis_index("s")
    w = s * NC + c

    def _zero(i, _):
        drain_v[pl.ds(i * 16, 16)] = jnp.zeros((16,), jnp.float32)
        return ()
    lax.fori_loop(0, RPT // 16, _zero, ())
    for j in range(CHUNK // 16):
        ones_v[pl.ds(j * 16, 16)] = jnp.ones((16,), jnp.float32)
    pltpu.sync_copy(drain_v, acc_s.at[pl.ds(s * RPT, RPT)])
    pltpu.sync_copy(drain_v, acc_d.at[pl.ds(s * RPT, RPT)])
    plsc.subcore_barrier()

    pltpu.sync_copy(srcm.at[pl.ds(w * CPT, CPT)], src_v)
    pltpu.sync_copy(dstm.at[pl.ds(w * CPT, CPT)], dst_v)

    def _body(g, _):
        pltpu.sync_copy(ones_v, acc_s.at[src_v.at[g]], add=True)
        pltpu.sync_copy(ones_v, acc_d.at[dst_v.at[g]], add=True)
        return ()
    lax.fori_loop(0, CPT, _body, ())

    plsc.subcore_barrier()
    pltpu.sync_copy(acc_s.at[pl.ds(s * RPT, RPT)], drain_v)
    pltpu.sync_copy(drain_v, out_s.at[pl.ds(c * NPAD + s * RPT, RPT)])
    pltpu.sync_copy(acc_d.at[pl.ds(s * RPT, RPT)], drain_v)
    pltpu.sync_copy(drain_v, out_d.at[pl.ds(c * NPAD + s * RPT, RPT)])


def _scatter_body(xn, srcm, dstm, out, src_v, dst_v, rows_v, acc, sem):
    c = lax.axis_index("c")
    s = lax.axis_index("s")
    w = s * NC + c

    def _zrow(i, _):
        for j in range(D // 16):
            rows_v[i, pl.ds(j * 16, 16)] = jnp.zeros((16,), jnp.float32)
        return ()
    lax.fori_loop(0, CHUNK, _zrow, ())
    for k in range(RPT // CHUNK):
        pltpu.sync_copy(rows_v, acc.at[pl.ds(s * RPT + k * CHUNK, CHUNK)])
    plsc.subcore_barrier()

    pltpu.sync_copy(srcm.at[pl.ds(w * CPT, CPT)], src_v)
    pltpu.sync_copy(dstm.at[pl.ds(w * CPT, CPT)], dst_v)

    def _body(g, _):
        pltpu.async_copy(xn.at[src_v.at[g]], rows_v, sem).wait()
        pltpu.sync_copy(rows_v, acc.at[dst_v.at[g]], add=True)
        return ()
    lax.fori_loop(0, CPT, _body, ())

    plsc.subcore_barrier()
    for k in range(RPT // CHUNK):
        r0 = s * RPT + k * CHUNK
        pltpu.sync_copy(acc.at[pl.ds(r0, CHUNK)], rows_v)
        pltpu.sync_copy(rows_v, out.at[c, pl.ds(r0, CHUNK)])


@functools.cache
def _sc_kernels():
    mesh = plsc.VectorSubcoreMesh(core_axis_name="c", subcore_axis_name="s")
    count_kernel = pl.kernel(
        _count_body,
        out_type=[
            jax.ShapeDtypeStruct((NC * NPAD,), jnp.float32),
            jax.ShapeDtypeStruct((NC * NPAD,), jnp.float32),
        ],
        mesh=mesh,
        scratch_types=[
            pltpu.VMEM((CPT, CHUNK), jnp.int32),
            pltpu.VMEM((CPT, CHUNK), jnp.int32),
            pltpu.VMEM((CHUNK,), jnp.float32),
            pltpu.VMEM((RPT,), jnp.float32),
            pltpu.VMEM_SHARED((NPAD,), jnp.float32),
            pltpu.VMEM_SHARED((NPAD,), jnp.float32),
        ],
    )
    scatter_kernel = pl.kernel(
        _scatter_body,
        out_type=jax.ShapeDtypeStruct((NC, NPAD, D), jnp.float32),
        mesh=mesh,
        scratch_types=[
            pltpu.VMEM((CPT, CHUNK), jnp.int32),
            pltpu.VMEM((CPT, CHUNK), jnp.int32),
            pltpu.VMEM((CHUNK, D), jnp.float32),
            pltpu.VMEM_SHARED((NPAD, D), jnp.float32),
            pltpu.SemaphoreType.DMA,
        ],
    )
    return count_kernel, scatter_kernel


def _norm_body(x_ref, cs_ref, xn_ref):
    deg_in = cs_ref[0] + cs_ref[1] + 1.0
    xn_ref[...] = x_ref[...] * lax.rsqrt(deg_in)[:, None]


_norm_call = pl.pallas_call(
    _norm_body,
    grid=(NPAD // BN,),
    in_specs=[
        pl.BlockSpec((BN, D), lambda i: (i, 0)),
        pl.BlockSpec((NC, BN), lambda i: (0, i)),
    ],
    out_specs=pl.BlockSpec((BN, D), lambda i: (i, 0)),
    out_shape=jax.ShapeDtypeStruct((NPAD, D), jnp.float32),
)


def _layer_body(p_ref, xn_ref, cs_ref, cd_ref, w_ref, b_ref,
                h_ref, hn_ref, g_ref):
    i = pl.program_id(0)
    deg_out = cd_ref[0] + cd_ref[1] + 1.0
    deg_in = cs_ref[0] + cs_ref[1] + 1.0
    upd = (p_ref[0] + p_ref[1] + xn_ref[...]) * lax.rsqrt(deg_out)[:, None]
    h = jnp.dot(upd, w_ref[...], preferred_element_type=jnp.float32)
    h = jnp.maximum(h + b_ref[...], 0.0)
    h_ref[...] = h
    hn_ref[...] = h * lax.rsqrt(deg_in)[:, None]
    rows = jax.lax.broadcasted_iota(jnp.int32, (BN, 1), 0) + i * BN
    bsum = jnp.sum(jnp.where(rows < N, h, 0.0), axis=0, keepdims=True)

    @pl.when(i == 0)
    def _():
        g_ref[...] = bsum

    @pl.when(i > 0)
    def _():
        g_ref[...] = g_ref[...] + bsum


_layer_call = pl.pallas_call(
    _layer_body,
    grid=(NPAD // BN,),
    in_specs=[
        pl.BlockSpec((NC, BN, D), lambda i: (0, i, 0)),
        pl.BlockSpec((BN, D), lambda i: (i, 0)),
        pl.BlockSpec((NC, BN), lambda i: (0, i)),
        pl.BlockSpec((NC, BN), lambda i: (0, i)),
        pl.BlockSpec((D, D), lambda i: (0, 0)),
        pl.BlockSpec((1, D), lambda i: (0, 0)),
    ],
    out_specs=[
        pl.BlockSpec((BN, D), lambda i: (i, 0)),
        pl.BlockSpec((BN, D), lambda i: (i, 0)),
        pl.BlockSpec((1, D), lambda i: (0, 0)),
    ],
    out_shape=[
        jax.ShapeDtypeStruct((NPAD, D), jnp.float32),
        jax.ShapeDtypeStruct((NPAD, D), jnp.float32),
        jax.ShapeDtypeStruct((1, D), jnp.float32),
    ],
)


def kernel(x, edge_index, W1, b1, W2, b2):
    src = edge_index[0]
    dst = edge_index[1]
    pad = jnp.full((EP - E,), N, jnp.int32)
    srcm = jnp.concatenate([src, pad]).reshape(EP // CHUNK, CHUNK)
    dstm = jnp.concatenate([dst, pad]).reshape(EP // CHUNK, CHUNK)
    xpad = jnp.zeros((NPAD, D), jnp.float32).at[:N].set(x)

    count_kernel, scatter_kernel = _sc_kernels()
    cs, cd = count_kernel(srcm, dstm)
    cs = cs.reshape(NC, NPAD)
    cd = cd.reshape(NC, NPAD)
    xn = _norm_call(xpad, cs)
    p1 = scatter_kernel(xn, srcm, dstm)
    _, hn1, _ = _layer_call(p1, xn, cs, cd, W1, b1.reshape(1, D))
    p2 = scatter_kernel(hn1, srcm, dstm)
    h2, _, gsum = _layer_call(p2, hn1, cs, cd, W2, b2.reshape(1, D))
    return (gsum, h2[:N])


# speedup vs baseline: 7.5501x; 1.1416x over previous
import functools

import jax
import jax.numpy as jnp
from jax import lax
from jax.experimental import pallas as pl
from jax.experimental.pallas import tpu as pltpu
from jax.experimental.pallas import tpu_sc as plsc

N = 10000
D = 128
E = 320000

NC = 2
NS = 16
NW = NC * NS

CHUNK = 128
CPT = 80
EPT = CPT * CHUNK
EP = EPT * NW
NPAD = 10240
RPT = NPAD // NS
BN = 512

def _count_body(srcm, dstm, out_s, out_d, src_v, dst_v, ones_v, drain_v,
                acc_s, acc_d):
    c = lax.axis_index("c")
    s = lax.axis_index("s")
    w = s * NC + c

    def _zero(i, _):
        drain_v[pl.ds(i * 16, 16)] = jnp.zeros((16,), jnp.float32)
        return ()
    lax.fori_loop(0, RPT // 16, _zero, ())
    for j in range(CHUNK // 16):
        ones_v[pl.ds(j * 16, 16)] = jnp.ones((16,), jnp.float32)
    pltpu.sync_copy(drain_v, acc_s.at[pl.ds(s * RPT, RPT)])
    pltpu.sync_copy(drain_v, acc_d.at[pl.ds(s * RPT, RPT)])
    plsc.subcore_barrier()

    pltpu.sync_copy(srcm.at[pl.ds(w * CPT, CPT)], src_v)
    pltpu.sync_copy(dstm.at[pl.ds(w * CPT, CPT)], dst_v)

    def _body(g, _):
        pltpu.sync_copy(ones_v, acc_s.at[src_v.at[g]], add=True)
        pltpu.sync_copy(ones_v, acc_d.at[dst_v.at[g]], add=True)
        return ()
    lax.fori_loop(0, CPT, _body, ())

    plsc.subcore_barrier()
    pltpu.sync_copy(acc_s.at[pl.ds(s * RPT, RPT)], drain_v)
    pltpu.sync_copy(drain_v, out_s.at[pl.ds(c * NPAD + s * RPT, RPT)])
    pltpu.sync_copy(acc_d.at[pl.ds(s * RPT, RPT)], drain_v)
    pltpu.sync_copy(drain_v, out_d.at[pl.ds(c * NPAD + s * RPT, RPT)])


IBLK = 8
NBLK = CPT // IBLK


def _scatter_body(xn, srcm3, dstm3, out, sidx, didx, rows2, acc, g0, g1):
    gs = (g0, g1)
    c = lax.axis_index("c")
    s = lax.axis_index("s")
    w = s * NC + c

    def _zrow(i, _):
        for j in range(D // 16):
            rows2[0, i, pl.ds(j * 16, 16)] = jnp.zeros((16,), jnp.float32)
        return ()
    lax.fori_loop(0, CHUNK, _zrow, ())
    for k in range(RPT // CHUNK):
        pltpu.sync_copy(rows2.at[0], acc.at[pl.ds(s * RPT + k * CHUNK, CHUNK)])
    plsc.subcore_barrier()

    def _block(bi, _):
        base = w * NBLK + bi
        pltpu.sync_copy(srcm3.at[base], sidx)
        pltpu.sync_copy(dstm3.at[base], didx)
        for j in range(IBLK):
            pltpu.async_copy(xn.at[sidx.at[j]], rows2.at[0], gs[0]).wait()
            pltpu.sync_copy(rows2.at[0], acc.at[didx.at[j]], add=True)
        return ()
    lax.fori_loop(0, NBLK, _block, ())

    plsc.subcore_barrier()
    for k in range(RPT // CHUNK):
        r0 = s * RPT + k * CHUNK
        pltpu.sync_copy(acc.at[pl.ds(r0, CHUNK)], rows2.at[0])
        pltpu.sync_copy(rows2.at[0], out.at[c, pl.ds(r0, CHUNK)])


@functools.cache
def _sc_kernels():
    mesh = plsc.VectorSubcoreMesh(core_axis_name="c", subcore_axis_name="s")
    count_kernel = pl.kernel(
        _count_body,
        out_type=[
            jax.ShapeDtypeStruct((NC * NPAD,), jnp.float32),
            jax.ShapeDtypeStruct((NC * NPAD,), jnp.float32),
        ],
        mesh=mesh,
        scratch_types=[
            pltpu.VMEM((CPT, CHUNK), jnp.int32),
            pltpu.VMEM((CPT, CHUNK), jnp.int32),
            pltpu.VMEM((CHUNK,), jnp.float32),
            pltpu.VMEM((RPT,), jnp.float32),
            pltpu.VMEM_SHARED((NPAD,), jnp.float32),
            pltpu.VMEM_SHARED((NPAD,), jnp.float32),
        ],
    )
    scatter_kernel = pl.kernel(
        _scatter_body,
        out_type=jax.ShapeDtypeStruct((NC, NPAD, D), jnp.float32),
        mesh=mesh,
        scratch_types=[
            pltpu.VMEM((IBLK, CHUNK), jnp.int32),
            pltpu.VMEM((IBLK, CHUNK), jnp.int32),
            pltpu.VMEM((2, CHUNK, D), jnp.float32),
            pltpu.VMEM_SHARED((NPAD, D), jnp.float32),
            pltpu.SemaphoreType.DMA,
            pltpu.SemaphoreType.DMA,
        ],
    )
    return count_kernel, scatter_kernel


def _norm_body(x_ref, cs_ref, xn_ref):
    deg_in = cs_ref[0] + cs_ref[1] + 1.0
    xn_ref[...] = x_ref[...] * lax.rsqrt(deg_in)[:, None]


_norm_call = pl.pallas_call(
    _norm_body,
    grid=(NPAD // BN,),
    in_specs=[
        pl.BlockSpec((BN, D), lambda i: (i, 0)),
        pl.BlockSpec((NC, BN), lambda i: (0, i)),
    ],
    out_specs=pl.BlockSpec((BN, D), lambda i: (i, 0)),
    out_shape=jax.ShapeDtypeStruct((NPAD, D), jnp.float32),
)


def _layer_body(p_ref, xn_ref, cs_ref, cd_ref, w_ref, b_ref,
                h_ref, hn_ref, g_ref):
    i = pl.program_id(0)
    deg_out = cd_ref[0] + cd_ref[1] + 1.0
    deg_in = cs_ref[0] + cs_ref[1] + 1.0
    upd = (p_ref[0] + p_ref[1] + xn_ref[...]) * lax.rsqrt(deg_out)[:, None]
    h = jnp.dot(upd, w_ref[...], preferred_element_type=jnp.float32)
    h = jnp.maximum(h + b_ref[...], 0.0)
    h_ref[...] = h
    hn_ref[...] = h * lax.rsqrt(deg_in)[:, None]
    rows = jax.lax.broadcasted_iota(jnp.int32, (BN, 1), 0) + i * BN
    bsum = jnp.sum(jnp.where(rows < N, h, 0.0), axis=0, keepdims=True)

    @pl.when(i == 0)
    def _():
        g_ref[...] = bsum

    @pl.when(i > 0)
    def _():
        g_ref[...] = g_ref[...] + bsum


_layer_call = pl.pallas_call(
    _layer_body,
    grid=(NPAD // BN,),
    in_specs=[
        pl.BlockSpec((NC, BN, D), lambda i: (0, i, 0)),
        pl.BlockSpec((BN, D), lambda i: (i, 0)),
        pl.BlockSpec((NC, BN), lambda i: (0, i)),
        pl.BlockSpec((NC, BN), lambda i: (0, i)),
        pl.BlockSpec((D, D), lambda i: (0, 0)),
        pl.BlockSpec((1, D), lambda i: (0, 0)),
    ],
    out_specs=[
        pl.BlockSpec((BN, D), lambda i: (i, 0)),
        pl.BlockSpec((BN, D), lambda i: (i, 0)),
        pl.BlockSpec((1, D), lambda i: (0, 0)),
    ],
    out_shape=[
        jax.ShapeDtypeStruct((NPAD, D), jnp.float32),
        jax.ShapeDtypeStruct((NPAD, D), jnp.float32),
        jax.ShapeDtypeStruct((1, D), jnp.float32),
    ],
)


def kernel(x, edge_index, W1, b1, W2, b2):
    src = edge_index[0]
    dst = edge_index[1]
    pad = jnp.full((EP - E,), N, jnp.int32)
    srcm = jnp.concatenate([src, pad]).reshape(EP // CHUNK, CHUNK)
    dstm = jnp.concatenate([dst, pad]).reshape(EP // CHUNK, CHUNK)
    xpad = jnp.zeros((NPAD, D), jnp.float32).at[:N].set(x)
    count_kernel, scatter_kernel = _sc_kernels()
    cs, cd = count_kernel(srcm, dstm)
    cs = cs.reshape(NC, NPAD)
    cd = cd.reshape(NC, NPAD)
    xn = _norm_call(xpad, cs)
    srcm3 = srcm.reshape(NW * NBLK, IBLK, CHUNK)
    dstm3 = dstm.reshape(NW * NBLK, IBLK, CHUNK)
    p1 = scatter_kernel(xn, srcm3, dstm3)
    _, hn1, _ = _layer_call(p1, xn, cs, cd, W1, b1.reshape(1, D))
    p2 = scatter_kernel(hn1, srcm3, dstm3)
    h2, _, gsum = _layer_call(p2, hn1, cs, cd, W2, b2.reshape(1, D))
    return (gsum, h2[:N])


# speedup vs baseline: 8.1694x; 1.0820x over previous
import functools

import jax
import jax.numpy as jnp
from jax import lax
from jax.experimental import pallas as pl
from jax.experimental.pallas import tpu as pltpu
from jax.experimental.pallas import tpu_sc as plsc

N = 10000
D = 128
E = 320000

NC = 2
NS = 16
NW = NC * NS

CHUNK = 128
CPT = 80
EPT = CPT * CHUNK
EP = EPT * NW
NPAD = 10240
RPT = NPAD // NS
BN = 512

def _count_body(srcm, dstm, out_s, out_d, src_v, dst_v, ones_v, drain_v,
                acc_s, acc_d):
    c = lax.axis_index("c")
    s = lax.axis_index("s")
    w = s * NC + c

    def _zero(i, _):
        drain_v[pl.ds(i * 16, 16)] = jnp.zeros((16,), jnp.float32)
        return ()
    lax.fori_loop(0, RPT // 16, _zero, ())
    for j in range(CHUNK // 16):
        ones_v[pl.ds(j * 16, 16)] = jnp.ones((16,), jnp.float32)
    pltpu.sync_copy(drain_v, acc_s.at[pl.ds(s * RPT, RPT)])
    pltpu.sync_copy(drain_v, acc_d.at[pl.ds(s * RPT, RPT)])
    plsc.subcore_barrier()

    pltpu.sync_copy(srcm.at[pl.ds(w * CPT, CPT)], src_v)
    pltpu.sync_copy(dstm.at[pl.ds(w * CPT, CPT)], dst_v)

    def _body(g, _):
        pltpu.sync_copy(ones_v, acc_s.at[src_v.at[g]], add=True)
        pltpu.sync_copy(ones_v, acc_d.at[dst_v.at[g]], add=True)
        return ()
    lax.fori_loop(0, CPT, _body, ())

    plsc.subcore_barrier()
    pltpu.sync_copy(acc_s.at[pl.ds(s * RPT, RPT)], drain_v)
    pltpu.sync_copy(drain_v, out_s.at[pl.ds(c * NPAD + s * RPT, RPT)])
    pltpu.sync_copy(acc_d.at[pl.ds(s * RPT, RPT)], drain_v)
    pltpu.sync_copy(drain_v, out_d.at[pl.ds(c * NPAD + s * RPT, RPT)])


IBLK = 8
NBLK = CPT // IBLK


def _scatter_body(xn, srcm3, dstm3, out, sidx, didx, rows2, acc, g0, g1):
    gs = (g0, g1)
    c = lax.axis_index("c")
    s = lax.axis_index("s")
    w = s * NC + c

    def _zrow(i, _):
        for j in range(D // 16):
            rows2[0, i, pl.ds(j * 16, 16)] = jnp.zeros((16,), jnp.float32)
        return ()
    lax.fori_loop(0, CHUNK, _zrow, ())
    for k in range(RPT // CHUNK):
        pltpu.sync_copy(rows2.at[0], acc.at[pl.ds(s * RPT + k * CHUNK, CHUNK)])
    plsc.subcore_barrier()

    def _block(bi, _):
        base = w * NBLK + bi
        pltpu.sync_copy(srcm3.at[base], sidx)
        pltpu.sync_copy(dstm3.at[base], didx)
        pltpu.async_copy(xn.at[sidx.at[0]], rows2.at[0], gs[0]).wait()
        for j in range(IBLK):
            if j < IBLK - 1:
                desc = pltpu.async_copy(
                    xn.at[sidx.at[j + 1]], rows2.at[(j + 1) % 2], gs[1])
            pltpu.sync_copy(rows2.at[j % 2], acc.at[didx.at[j]], add=True)
            if j < IBLK - 1:
                desc.wait()
        return ()
    lax.fori_loop(0, NBLK, _block, ())

    plsc.subcore_barrier()
    for k in range(RPT // CHUNK):
        r0 = s * RPT + k * CHUNK
        pltpu.sync_copy(acc.at[pl.ds(r0, CHUNK)], rows2.at[0])
        pltpu.sync_copy(rows2.at[0], out.at[c, pl.ds(r0, CHUNK)])


@functools.cache
def _sc_kernels():
    mesh = plsc.VectorSubcoreMesh(core_axis_name="c", subcore_axis_name="s")
    count_kernel = pl.kernel(
        _count_body,
        out_type=[
            jax.ShapeDtypeStruct((NC * NPAD,), jnp.float32),
            jax.ShapeDtypeStruct((NC * NPAD,), jnp.float32),
        ],
        mesh=mesh,
        scratch_types=[
            pltpu.VMEM((CPT, CHUNK), jnp.int32),
            pltpu.VMEM((CPT, CHUNK), jnp.int32),
            pltpu.VMEM((CHUNK,), jnp.float32),
            pltpu.VMEM((RPT,), jnp.float32),
            pltpu.VMEM_SHARED((NPAD,), jnp.float32),
            pltpu.VMEM_SHARED((NPAD,), jnp.float32),
        ],
    )
    scatter_kernel = pl.kernel(
        _scatter_body,
        out_type=jax.ShapeDtypeStruct((NC, NPAD, D), jnp.float32),
        mesh=mesh,
        scratch_types=[
            pltpu.VMEM((IBLK, CHUNK), jnp.int32),
            pltpu.VMEM((IBLK, CHUNK), jnp.int32),
            pltpu.VMEM((2, CHUNK, D), jnp.float32),
            pltpu.VMEM_SHARED((NPAD, D), jnp.float32),
            pltpu.SemaphoreType.DMA,
            pltpu.SemaphoreType.DMA,
        ],
    )
    return count_kernel, scatter_kernel


def _norm_body(x_ref, cs_ref, xn_ref):
    deg_in = cs_ref[0] + cs_ref[1] + 1.0
    xn_ref[...] = x_ref[...] * lax.rsqrt(deg_in)[:, None]


_norm_call = pl.pallas_call(
    _norm_body,
    grid=(NPAD // BN,),
    in_specs=[
        pl.BlockSpec((BN, D), lambda i: (i, 0)),
        pl.BlockSpec((NC, BN), lambda i: (0, i)),
    ],
    out_specs=pl.BlockSpec((BN, D), lambda i: (i, 0)),
    out_shape=jax.ShapeDtypeStruct((NPAD, D), jnp.float32),
)


def _layer_body(p_ref, xn_ref, cs_ref, cd_ref, w_ref, b_ref,
                h_ref, hn_ref, g_ref):
    i = pl.program_id(0)
    deg_out = cd_ref[0] + cd_ref[1] + 1.0
    deg_in = cs_ref[0] + cs_ref[1] + 1.0
    upd = (p_ref[0] + p_ref[1] + xn_ref[...]) * lax.rsqrt(deg_out)[:, None]
    h = jnp.dot(upd, w_ref[...], preferred_element_type=jnp.float32)
    h = jnp.maximum(h + b_ref[...], 0.0)
    h_ref[...] = h
    hn_ref[...] = h * lax.rsqrt(deg_in)[:, None]
    rows = jax.lax.broadcasted_iota(jnp.int32, (BN, 1), 0) + i * BN
    bsum = jnp.sum(jnp.where(rows < N, h, 0.0), axis=0, keepdims=True)

    @pl.when(i == 0)
    def _():
        g_ref[...] = bsum

    @pl.when(i > 0)
    def _():
        g_ref[...] = g_ref[...] + bsum


_layer_call = pl.pallas_call(
    _layer_body,
    grid=(NPAD // BN,),
    in_specs=[
        pl.BlockSpec((NC, BN, D), lambda i: (0, i, 0)),
        pl.BlockSpec((BN, D), lambda i: (i, 0)),
        pl.BlockSpec((NC, BN), lambda i: (0, i)),
        pl.BlockSpec((NC, BN), lambda i: (0, i)),
        pl.BlockSpec((D, D), lambda i: (0, 0)),
        pl.BlockSpec((1, D), lambda i: (0, 0)),
    ],
    out_specs=[
        pl.BlockSpec((BN, D), lambda i: (i, 0)),
        pl.BlockSpec((BN, D), lambda i: (i, 0)),
        pl.BlockSpec((1, D), lambda i: (0, 0)),
    ],
    out_shape=[
        jax.ShapeDtypeStruct((NPAD, D), jnp.float32),
        jax.ShapeDtypeStruct((NPAD, D), jnp.float32),
        jax.ShapeDtypeStruct((1, D), jnp.float32),
    ],
)


def kernel(x, edge_index, W1, b1, W2, b2):
    src = edge_index[0]
    dst = edge_index[1]
    pad = jnp.full((EP - E,), N, jnp.int32)
    srcm = jnp.concatenate([src, pad]).reshape(EP // CHUNK, CHUNK)
    dstm = jnp.concatenate([dst, pad]).reshape(EP // CHUNK, CHUNK)
    xpad = jnp.zeros((NPAD, D), jnp.float32).at[:N].set(x)
    count_kernel, scatter_kernel = _sc_kernels()
    cs, cd = count_kernel(srcm, dstm)
    cs = cs.reshape(NC, NPAD)
    cd = cd.reshape(NC, NPAD)
    xn = _norm_call(xpad, cs)
    srcm3 = srcm.reshape(NW * NBLK, IBLK, CHUNK)
    dstm3 = dstm.reshape(NW * NBLK, IBLK, CHUNK)
    p1 = scatter_kernel(xn, srcm3, dstm3)
    _, hn1, _ = _layer_call(p1, xn, cs, cd, W1, b1.reshape(1, D))
    p2 = scatter_kernel(hn1, srcm3, dstm3)
    h2, _, gsum = _layer_call(p2, hn1, cs, cd, W2, b2.reshape(1, D))
    return (gsum, h2[:N])


# speedup vs baseline: 8.8014x; 1.0774x over previous
import functools

import jax
import jax.numpy as jnp
from jax import lax
from jax.experimental import pallas as pl
from jax.experimental.pallas import tpu as pltpu
from jax.experimental.pallas import tpu_sc as plsc

N = 10000
D = 128
E = 320000

NC = 2
NS = 16
NW = NC * NS

CHUNK = 128
CPT = 80
EPT = CPT * CHUNK
EP = EPT * NW
NPAD = 10240
RPT = NPAD // NS
BN = 512

def _count_body(srcm, dstm, out_s, out_d, src_v, dst_v, ones_v, drain_v,
                acc_s, acc_d):
    c = lax.axis_index("c")
    s = lax.axis_index("s")
    w = s * NC + c

    def _zero(i, _):
        drain_v[pl.ds(i * 16, 16)] = jnp.zeros((16,), jnp.float32)
        return ()
    lax.fori_loop(0, RPT // 16, _zero, ())
    for j in range(CHUNK // 16):
        ones_v[pl.ds(j * 16, 16)] = jnp.ones((16,), jnp.float32)
    pltpu.sync_copy(drain_v, acc_s.at[pl.ds(s * RPT, RPT)])
    pltpu.sync_copy(drain_v, acc_d.at[pl.ds(s * RPT, RPT)])
    plsc.subcore_barrier()

    pltpu.sync_copy(srcm.at[pl.ds(w * CPT, CPT)], src_v)
    pltpu.sync_copy(dstm.at[pl.ds(w * CPT, CPT)], dst_v)

    def _body(g, _):
        pltpu.sync_copy(ones_v, acc_s.at[src_v.at[g]], add=True)
        pltpu.sync_copy(ones_v, acc_d.at[dst_v.at[g]], add=True)
        return ()
    lax.fori_loop(0, CPT, _body, ())

    plsc.subcore_barrier()
    pltpu.sync_copy(acc_s.at[pl.ds(s * RPT, RPT)], drain_v)
    pltpu.sync_copy(drain_v, out_s.at[pl.ds(c * NPAD + s * RPT, RPT)])
    pltpu.sync_copy(acc_d.at[pl.ds(s * RPT, RPT)], drain_v)
    pltpu.sync_copy(drain_v, out_d.at[pl.ds(c * NPAD + s * RPT, RPT)])


IBLK = 8
NBLK = CPT // IBLK
NBLK0 = 14
NBLK1 = 6


def _scatter_body(xn, srcm3, dstm3, out, sidx, didx, rows2, acc, g0, g1):
    gs = (g0, g1)
    c = lax.axis_index("c")
    s = lax.axis_index("s")
    w = s * NC + c

    def _zrow(i, _):
        for j in range(D // 16):
            rows2[0, i, pl.ds(j * 16, 16)] = jnp.zeros((16,), jnp.float32)
        return ()
    lax.fori_loop(0, CHUNK, _zrow, ())
    for k in range(RPT // CHUNK):
        pltpu.sync_copy(rows2.at[0], acc.at[pl.ds(s * RPT + k * CHUNK, CHUNK)])
    plsc.subcore_barrier()

    nblk = jnp.where(c == 0, NBLK0, NBLK1)
    blk0 = jnp.where(c == 0, s * NBLK0, NS * NBLK0 + s * NBLK1)

    def _block(bi, _):
        base = blk0 + bi
        pltpu.sync_copy(srcm3.at[base], sidx)
        pltpu.sync_copy(dstm3.at[base], didx)
        pltpu.async_copy(xn.at[sidx.at[0]], rows2.at[0], gs[0]).wait()
        for j in range(IBLK):
            if j < IBLK - 1:
                desc = pltpu.async_copy(
                    xn.at[sidx.at[j + 1]], rows2.at[(j + 1) % 2], gs[1])
            pltpu.sync_copy(rows2.at[j % 2], acc.at[didx.at[j]], add=True)
            if j < IBLK - 1:
                desc.wait()
        return ()
    lax.fori_loop(0, nblk, _block, ())

    plsc.subcore_barrier()
    for k in range(RPT // CHUNK):
        r0 = s * RPT + k * CHUNK
        pltpu.sync_copy(acc.at[pl.ds(r0, CHUNK)], rows2.at[0])
        pltpu.sync_copy(rows2.at[0], out.at[c, pl.ds(r0, CHUNK)])


@functools.cache
def _sc_kernels():
    mesh = plsc.VectorSubcoreMesh(core_axis_name="c", subcore_axis_name="s")
    count_kernel = pl.kernel(
        _count_body,
        out_type=[
            jax.ShapeDtypeStruct((NC * NPAD,), jnp.float32),
            jax.ShapeDtypeStruct((NC * NPAD,), jnp.float32),
        ],
        mesh=mesh,
        scratch_types=[
            pltpu.VMEM((CPT, CHUNK), jnp.int32),
            pltpu.VMEM((CPT, CHUNK), jnp.int32),
            pltpu.VMEM((CHUNK,), jnp.float32),
            pltpu.VMEM((RPT,), jnp.float32),
            pltpu.VMEM_SHARED((NPAD,), jnp.float32),
            pltpu.VMEM_SHARED((NPAD,), jnp.float32),
        ],
    )
    scatter_kernel = pl.kernel(
        _scatter_body,
        out_type=jax.ShapeDtypeStruct((NC, NPAD, D), jnp.float32),
        mesh=mesh,
        scratch_types=[
            pltpu.VMEM((IBLK, CHUNK), jnp.int32),
            pltpu.VMEM((IBLK, CHUNK), jnp.int32),
            pltpu.VMEM((2, CHUNK, D), jnp.float32),
            pltpu.VMEM_SHARED((NPAD, D), jnp.float32),
            pltpu.SemaphoreType.DMA,
            pltpu.SemaphoreType.DMA,
        ],
    )
    return count_kernel, scatter_kernel


def _norm_body(x_ref, cs_ref, xn_ref):
    deg_in = cs_ref[0] + cs_ref[1] + 1.0
    xn_ref[...] = x_ref[...] * lax.rsqrt(deg_in)[:, None]


_norm_call = pl.pallas_call(
    _norm_body,
    grid=(NPAD // BN,),
    in_specs=[
        pl.BlockSpec((BN, D), lambda i: (i, 0)),
        pl.BlockSpec((NC, BN), lambda i: (0, i)),
    ],
    out_specs=pl.BlockSpec((BN, D), lambda i: (i, 0)),
    out_shape=jax.ShapeDtypeStruct((NPAD, D), jnp.float32),
)


def _layer_body(p_ref, xn_ref, cs_ref, cd_ref, w_ref, b_ref,
                h_ref, hn_ref, g_ref):
    i = pl.program_id(0)
    deg_out = cd_ref[0] + cd_ref[1] + 1.0
    deg_in = cs_ref[0] + cs_ref[1] + 1.0
    upd = (p_ref[0] + p_ref[1] + xn_ref[...]) * lax.rsqrt(deg_out)[:, None]
    h = jnp.dot(upd, w_ref[...], preferred_element_type=jnp.float32)
    h = jnp.maximum(h + b_ref[...], 0.0)
    h_ref[...] = h
    hn_ref[...] = h * lax.rsqrt(deg_in)[:, None]
    rows = jax.lax.broadcasted_iota(jnp.int32, (BN, 1), 0) + i * BN
    bsum = jnp.sum(jnp.where(rows < N, h, 0.0), axis=0, keepdims=True)

    @pl.when(i == 0)
    def _():
        g_ref[...] = bsum

    @pl.when(i > 0)
    def _():
        g_ref[...] = g_ref[...] + bsum


_layer_call = pl.pallas_call(
    _layer_body,
    grid=(NPAD // BN,),
    in_specs=[
        pl.BlockSpec((NC, BN, D), lambda i: (0, i, 0)),
        pl.BlockSpec((BN, D), lambda i: (i, 0)),
        pl.BlockSpec((NC, BN), lambda i: (0, i)),
        pl.BlockSpec((NC, BN), lambda i: (0, i)),
        pl.BlockSpec((D, D), lambda i: (0, 0)),
        pl.BlockSpec((1, D), lambda i: (0, 0)),
    ],
    out_specs=[
        pl.BlockSpec((BN, D), lambda i: (i, 0)),
        pl.BlockSpec((BN, D), lambda i: (i, 0)),
        pl.BlockSpec((1, D), lambda i: (0, 0)),
    ],
    out_shape=[
        jax.ShapeDtypeStruct((NPAD, D), jnp.float32),
        jax.ShapeDtypeStruct((NPAD, D), jnp.float32),
        jax.ShapeDtypeStruct((1, D), jnp.float32),
    ],
)


def kernel(x, edge_index, W1, b1, W2, b2):
    src = edge_index[0]
    dst = edge_index[1]
    pad = jnp.full((EP - E,), N, jnp.int32)
    srcm = jnp.concatenate([src, pad]).reshape(EP // CHUNK, CHUNK)
    dstm = jnp.concatenate([dst, pad]).reshape(EP // CHUNK, CHUNK)
    xpad = jnp.zeros((NPAD, D), jnp.float32).at[:N].set(x)
    count_kernel, scatter_kernel = _sc_kernels()
    cs, cd = count_kernel(srcm, dstm)
    cs = cs.reshape(NC, NPAD)
    cd = cd.reshape(NC, NPAD)
    xn = _norm_call(xpad, cs)
    srcm3 = srcm.reshape(NW * NBLK, IBLK, CHUNK)
    dstm3 = dstm.reshape(NW * NBLK, IBLK, CHUNK)
    p1 = scatter_kernel(xn, srcm3, dstm3)
    _, hn1, _ = _layer_call(p1, xn, cs, cd, W1, b1.reshape(1, D))
    p2 = scatter_kernel(hn1, srcm3, dstm3)
    h2, _, gsum = _layer_call(p2, hn1, cs, cd, W2, b2.reshape(1, D))
    return (gsum, h2[:N])


# speedup vs baseline: 9.0882x; 1.0326x over previous
import functools

import jax
import jax.numpy as jnp
from jax import lax
from jax.experimental import pallas as pl
from jax.experimental.pallas import tpu as pltpu
from jax.experimental.pallas import tpu_sc as plsc

N = 10000
D = 128
E = 320000

NC = 2
NS = 16
NW = NC * NS

CHUNK = 128
CPT = 80
EPT = CPT * CHUNK
EP = EPT * NW
NPAD = 10240
RPT = NPAD // NS
BN = 512

def _count_body(srcm, dstm, out_s, out_d, src_v, dst_v, ones_v, drain_v,
                acc_s, acc_d):
    c = lax.axis_index("c")
    s = lax.axis_index("s")
    w = s * NC + c

    def _zero(i, _):
        drain_v[pl.ds(i * 16, 16)] = jnp.zeros((16,), jnp.float32)
        return ()
    lax.fori_loop(0, RPT // 16, _zero, ())
    for j in range(CHUNK // 16):
        ones_v[pl.ds(j * 16, 16)] = jnp.ones((16,), jnp.float32)
    pltpu.sync_copy(drain_v, acc_s.at[pl.ds(s * RPT, RPT)])
    pltpu.sync_copy(drain_v, acc_d.at[pl.ds(s * RPT, RPT)])
    plsc.subcore_barrier()

    pltpu.sync_copy(srcm.at[pl.ds(w * CPT, CPT)], src_v)
    pltpu.sync_copy(dstm.at[pl.ds(w * CPT, CPT)], dst_v)

    def _body(g, _):
        pltpu.sync_copy(ones_v, acc_s.at[src_v.at[g]], add=True)
        pltpu.sync_copy(ones_v, acc_d.at[dst_v.at[g]], add=True)
        return ()
    lax.fori_loop(0, CPT, _body, ())

    plsc.subcore_barrier()
    pltpu.sync_copy(acc_s.at[pl.ds(s * RPT, RPT)], drain_v)
    pltpu.sync_copy(drain_v, out_s.at[pl.ds(c * NPAD + s * RPT, RPT)])
    pltpu.sync_copy(acc_d.at[pl.ds(s * RPT, RPT)], drain_v)
    pltpu.sync_copy(drain_v, out_d.at[pl.ds(c * NPAD + s * RPT, RPT)])


IBLK = 8
NBLK = CPT // IBLK
NBLK0 = 16
NBLK1 = 4


def _scatter_body(xn, srcm3, dstm3, out, sidx, didx, rows2, acc, g0, g1):
    gs = (g0, g1)
    c = lax.axis_index("c")
    s = lax.axis_index("s")
    w = s * NC + c

    def _zrow(i, _):
        for j in range(D // 16):
            rows2[0, i, pl.ds(j * 16, 16)] = jnp.zeros((16,), jnp.float32)
        return ()
    lax.fori_loop(0, CHUNK, _zrow, ())
    for k in range(RPT // CHUNK):
        pltpu.sync_copy(rows2.at[0], acc.at[pl.ds(s * RPT + k * CHUNK, CHUNK)])
    plsc.subcore_barrier()

    nblk = jnp.where(c == 0, NBLK0, NBLK1)
    blk0 = jnp.where(c == 0, s * NBLK0, NS * NBLK0 + s * NBLK1)

    def _block(bi, _):
        base = blk0 + bi
        pltpu.sync_copy(srcm3.at[base], sidx)
        pltpu.sync_copy(dstm3.at[base], didx)
        pltpu.async_copy(xn.at[sidx.at[0]], rows2.at[0], gs[0]).wait()
        for j in range(IBLK):
            if j < IBLK - 1:
                desc = pltpu.async_copy(
                    xn.at[sidx.at[j + 1]], rows2.at[(j + 1) % 2], gs[1])
            pltpu.sync_copy(rows2.at[j % 2], acc.at[didx.at[j]], add=True)
            if j < IBLK - 1:
                desc.wait()
        return ()
    lax.fori_loop(0, nblk, _block, ())

    plsc.subcore_barrier()
    for k in range(RPT // CHUNK):
        r0 = s * RPT + k * CHUNK
        pltpu.sync_copy(acc.at[pl.ds(r0, CHUNK)], rows2.at[0])
        pltpu.sync_copy(rows2.at[0], out.at[c, pl.ds(r0, CHUNK)])


@functools.cache
def _sc_kernels():
    mesh = plsc.VectorSubcoreMesh(core_axis_name="c", subcore_axis_name="s")
    count_kernel = pl.kernel(
        _count_body,
        out_type=[
            jax.ShapeDtypeStruct((NC * NPAD,), jnp.float32),
            jax.ShapeDtypeStruct((NC * NPAD,), jnp.float32),
        ],
        mesh=mesh,
        scratch_types=[
            pltpu.VMEM((CPT, CHUNK), jnp.int32),
            pltpu.VMEM((CPT, CHUNK), jnp.int32),
            pltpu.VMEM((CHUNK,), jnp.float32),
            pltpu.VMEM((RPT,), jnp.float32),
            pltpu.VMEM_SHARED((NPAD,), jnp.float32),
            pltpu.VMEM_SHARED((NPAD,), jnp.float32),
        ],
    )
    scatter_kernel = pl.kernel(
        _scatter_body,
        out_type=jax.ShapeDtypeStruct((NC, NPAD, D), jnp.float32),
        mesh=mesh,
        scratch_types=[
            pltpu.VMEM((IBLK, CHUNK), jnp.int32),
            pltpu.VMEM((IBLK, CHUNK), jnp.int32),
            pltpu.VMEM((2, CHUNK, D), jnp.float32),
            pltpu.VMEM_SHARED((NPAD, D), jnp.float32),
            pltpu.SemaphoreType.DMA,
            pltpu.SemaphoreType.DMA,
        ],
    )
    return count_kernel, scatter_kernel


def _norm_body(x_ref, cs_ref, xn_ref):
    deg_in = cs_ref[0] + cs_ref[1] + 1.0
    xn_ref[...] = x_ref[...] * lax.rsqrt(deg_in)[:, None]


_norm_call = pl.pallas_call(
    _norm_body,
    grid=(NPAD // BN,),
    in_specs=[
        pl.BlockSpec((BN, D), lambda i: (i, 0)),
        pl.BlockSpec((NC, BN), lambda i: (0, i)),
    ],
    out_specs=pl.BlockSpec((BN, D), lambda i: (i, 0)),
    out_shape=jax.ShapeDtypeStruct((NPAD, D), jnp.float32),
)


def _layer_body(p_ref, xn_ref, cs_ref, cd_ref, w_ref, b_ref,
                h_ref, hn_ref, g_ref):
    i = pl.program_id(0)
    deg_out = cd_ref[0] + cd_ref[1] + 1.0
    deg_in = cs_ref[0] + cs_ref[1] + 1.0
    upd = (p_ref[0] + p_ref[1] + xn_ref[...]) * lax.rsqrt(deg_out)[:, None]
    h = jnp.dot(upd, w_ref[...], preferred_element_type=jnp.float32)
    h = jnp.maximum(h + b_ref[...], 0.0)
    h_ref[...] = h
    hn_ref[...] = h * lax.rsqrt(deg_in)[:, None]
    rows = jax.lax.broadcasted_iota(jnp.int32, (BN, 1), 0) + i * BN
    bsum = jnp.sum(jnp.where(rows < N, h, 0.0), axis=0, keepdims=True)

    @pl.when(i == 0)
    def _():
        g_ref[...] = bsum

    @pl.when(i > 0)
    def _():
        g_ref[...] = g_ref[...] + bsum


_layer_call = pl.pallas_call(
    _layer_body,
    grid=(NPAD // BN,),
    in_specs=[
        pl.BlockSpec((NC, BN, D), lambda i: (0, i, 0)),
        pl.BlockSpec((BN, D), lambda i: (i, 0)),
        pl.BlockSpec((NC, BN), lambda i: (0, i)),
        pl.BlockSpec((NC, BN), lambda i: (0, i)),
        pl.BlockSpec((D, D), lambda i: (0, 0)),
        pl.BlockSpec((1, D), lambda i: (0, 0)),
    ],
    out_specs=[
        pl.BlockSpec((BN, D), lambda i: (i, 0)),
        pl.BlockSpec((BN, D), lambda i: (i, 0)),
        pl.BlockSpec((1, D), lambda i: (0, 0)),
    ],
    out_shape=[
        jax.ShapeDtypeStruct((NPAD, D), jnp.float32),
        jax.ShapeDtypeStruct((NPAD, D), jnp.float32),
        jax.ShapeDtypeStruct((1, D), jnp.float32),
    ],
)


def kernel(x, edge_index, W1, b1, W2, b2):
    src = edge_index[0]
    dst = edge_index[1]
    pad = jnp.full((EP - E,), N, jnp.int32)
    srcm = jnp.concatenate([src, pad]).reshape(EP // CHUNK, CHUNK)
    dstm = jnp.concatenate([dst, pad]).reshape(EP // CHUNK, CHUNK)
    xpad = jnp.zeros((NPAD, D), jnp.float32).at[:N].set(x)
    count_kernel, scatter_kernel = _sc_kernels()
    cs, cd = count_kernel(srcm, dstm)
    cs = cs.reshape(NC, NPAD)
    cd = cd.reshape(NC, NPAD)
    xn = _norm_call(xpad, cs)
    srcm3 = srcm.reshape(NW * NBLK, IBLK, CHUNK)
    dstm3 = dstm.reshape(NW * NBLK, IBLK, CHUNK)
    p1 = scatter_kernel(xn, srcm3, dstm3)
    _, hn1, _ = _layer_call(p1, xn, cs, cd, W1, b1.reshape(1, D))
    p2 = scatter_kernel(hn1, srcm3, dstm3)
    h2, _, gsum = _layer_call(p2, hn1, cs, cd, W2, b2.reshape(1, D))
    return (gsum, h2[:N])


# speedup vs baseline: 9.1223x; 1.0038x over previous
import functools

import jax
import jax.numpy as jnp
from jax import lax
from jax.experimental import pallas as pl
from jax.experimental.pallas import tpu as pltpu
from jax.experimental.pallas import tpu_sc as plsc

N = 10000
D = 128
E = 320000

NC = 2
NS = 16
NW = NC * NS

CHUNK = 128
CPT = 80
EPT = CPT * CHUNK
EP = EPT * NW
NPAD = 10240
RPT = NPAD // NS
BN = 512

def _count_body(srcm, dstm, out_s, out_d, src_v, dst_v, ones_v, drain_v,
                acc_s, acc_d):
    c = lax.axis_index("c")
    s = lax.axis_index("s")
    w = s * NC + c

    def _zero(i, _):
        drain_v[pl.ds(i * 16, 16)] = jnp.zeros((16,), jnp.float32)
        return ()
    lax.fori_loop(0, RPT // 16, _zero, ())
    for j in range(CHUNK // 16):
        ones_v[pl.ds(j * 16, 16)] = jnp.ones((16,), jnp.float32)
    pltpu.sync_copy(drain_v, acc_s.at[pl.ds(s * RPT, RPT)])
    pltpu.sync_copy(drain_v, acc_d.at[pl.ds(s * RPT, RPT)])
    plsc.subcore_barrier()

    pltpu.sync_copy(srcm.at[pl.ds(w * CPT, CPT)], src_v)
    pltpu.sync_copy(dstm.at[pl.ds(w * CPT, CPT)], dst_v)

    def _body(g, _):
        pltpu.sync_copy(ones_v, acc_s.at[src_v.at[g]], add=True)
        pltpu.sync_copy(ones_v, acc_d.at[dst_v.at[g]], add=True)
        return ()
    lax.fori_loop(0, CPT, _body, ())

    plsc.subcore_barrier()
    pltpu.sync_copy(acc_s.at[pl.ds(s * RPT, RPT)], drain_v)
    pltpu.sync_copy(drain_v, out_s.at[pl.ds(c * NPAD + s * RPT, RPT)])
    pltpu.sync_copy(acc_d.at[pl.ds(s * RPT, RPT)], drain_v)
    pltpu.sync_copy(drain_v, out_d.at[pl.ds(c * NPAD + s * RPT, RPT)])


IBLK = 8
NBLK = CPT // IBLK
NBLK0 = 17
NBLK1 = 3


def _scatter_body(xn, srcm3, dstm3, out, sidx, didx, rows2, acc, g0, g1):
    gs = (g0, g1)
    c = lax.axis_index("c")
    s = lax.axis_index("s")
    w = s * NC + c

    def _zrow(i, _):
        for j in range(D // 16):
            rows2[0, i, pl.ds(j * 16, 16)] = jnp.zeros((16,), jnp.float32)
        return ()
    lax.fori_loop(0, CHUNK, _zrow, ())
    for k in range(RPT // CHUNK):
        pltpu.sync_copy(rows2.at[0], acc.at[pl.ds(s * RPT + k * CHUNK, CHUNK)])
    plsc.subcore_barrier()

    nblk = jnp.where(c == 0, NBLK0, NBLK1)
    blk0 = jnp.where(c == 0, s * NBLK0, NS * NBLK0 + s * NBLK1)

    def _block(bi, _):
        base = blk0 + bi
        pltpu.sync_copy(srcm3.at[base], sidx)
        pltpu.sync_copy(dstm3.at[base], didx)
        pltpu.async_copy(xn.at[sidx.at[0]], rows2.at[0], gs[0]).wait()
        for j in range(IBLK):
            if j < IBLK - 1:
                desc = pltpu.async_copy(
                    xn.at[sidx.at[j + 1]], rows2.at[(j + 1) % 2], gs[1])
            pltpu.sync_copy(rows2.at[j % 2], acc.at[didx.at[j]], add=True)
            if j < IBLK - 1:
                desc.wait()
        return ()
    lax.fori_loop(0, nblk, _block, ())

    plsc.subcore_barrier()
    for k in range(RPT // CHUNK):
        r0 = s * RPT + k * CHUNK
        pltpu.sync_copy(acc.at[pl.ds(r0, CHUNK)], rows2.at[0])
        pltpu.sync_copy(rows2.at[0], out.at[c, pl.ds(r0, CHUNK)])


@functools.cache
def _sc_kernels():
    mesh = plsc.VectorSubcoreMesh(core_axis_name="c", subcore_axis_name="s")
    count_kernel = pl.kernel(
        _count_body,
        out_type=[
            jax.ShapeDtypeStruct((NC * NPAD,), jnp.float32),
            jax.ShapeDtypeStruct((NC * NPAD,), jnp.float32),
        ],
        mesh=mesh,
        scratch_types=[
            pltpu.VMEM((CPT, CHUNK), jnp.int32),
            pltpu.VMEM((CPT, CHUNK), jnp.int32),
            pltpu.VMEM((CHUNK,), jnp.float32),
            pltpu.VMEM((RPT,), jnp.float32),
            pltpu.VMEM_SHARED((NPAD,), jnp.float32),
            pltpu.VMEM_SHARED((NPAD,), jnp.float32),
        ],
    )
    scatter_kernel = pl.kernel(
        _scatter_body,
        out_type=jax.ShapeDtypeStruct((NC, NPAD, D), jnp.float32),
        mesh=mesh,
        scratch_types=[
            pltpu.VMEM((IBLK, CHUNK), jnp.int32),
            pltpu.VMEM((IBLK, CHUNK), jnp.int32),
            pltpu.VMEM((2, CHUNK, D), jnp.float32),
            pltpu.VMEM_SHARED((NPAD, D), jnp.float32),
            pltpu.SemaphoreType.DMA,
            pltpu.SemaphoreType.DMA,
        ],
    )
    return count_kernel, scatter_kernel


def _norm_body(x_ref, cs_ref, xn_ref):
    deg_in = cs_ref[0] + cs_ref[1] + 1.0
    xn_ref[...] = x_ref[...] * lax.rsqrt(deg_in)[:, None]


_norm_call = pl.pallas_call(
    _norm_body,
    grid=(NPAD // BN,),
    in_specs=[
        pl.BlockSpec((BN, D), lambda i: (i, 0)),
        pl.BlockSpec((NC, BN), lambda i: (0, i)),
    ],
    out_specs=pl.BlockSpec((BN, D), lambda i: (i, 0)),
    out_shape=jax.ShapeDtypeStruct((NPAD, D), jnp.float32),
)


def _layer_body(p_ref, xn_ref, cs_ref, cd_ref, w_ref, b_ref,
                h_ref, hn_ref, g_ref):
    i = pl.program_id(0)
    deg_out = cd_ref[0] + cd_ref[1] + 1.0
    deg_in = cs_ref[0] + cs_ref[1] + 1.0
    upd = (p_ref[0] + p_ref[1] + xn_ref[...]) * lax.rsqrt(deg_out)[:, None]
    h = jnp.dot(upd, w_ref[...], preferred_element_type=jnp.float32)
    h = jnp.maximum(h + b_ref[...], 0.0)
    h_ref[...] = h
    hn_ref[...] = h * lax.rsqrt(deg_in)[:, None]
    rows = jax.lax.broadcasted_iota(jnp.int32, (BN, 1), 0) + i * BN
    bsum = jnp.sum(jnp.where(rows < N, h, 0.0), axis=0, keepdims=True)

    @pl.when(i == 0)
    def _():
        g_ref[...] = bsum

    @pl.when(i > 0)
    def _():
        g_ref[...] = g_ref[...] + bsum


_layer_call = pl.pallas_call(
    _layer_body,
    grid=(NPAD // BN,),
    in_specs=[
        pl.BlockSpec((NC, BN, D), lambda i: (0, i, 0)),
        pl.BlockSpec((BN, D), lambda i: (i, 0)),
        pl.BlockSpec((NC, BN), lambda i: (0, i)),
        pl.BlockSpec((NC, BN), lambda i: (0, i)),
        pl.BlockSpec((D, D), lambda i: (0, 0)),
        pl.BlockSpec((1, D), lambda i: (0, 0)),
    ],
    out_specs=[
        pl.BlockSpec((BN, D), lambda i: (i, 0)),
        pl.BlockSpec((BN, D), lambda i: (i, 0)),
        pl.BlockSpec((1, D), lambda i: (0, 0)),
    ],
    out_shape=[
        jax.ShapeDtypeStruct((NPAD, D), jnp.float32),
        jax.ShapeDtypeStruct((NPAD, D), jnp.float32),
        jax.ShapeDtypeStruct((1, D), jnp.float32),
    ],
)


def kernel(x, edge_index, W1, b1, W2, b2):
    src = edge_index[0]
    dst = edge_index[1]
    pad = jnp.full((EP - E,), N, jnp.int32)
    srcm = jnp.concatenate([src, pad]).reshape(EP // CHUNK, CHUNK)
    dstm = jnp.concatenate([dst, pad]).reshape(EP // CHUNK, CHUNK)
    xpad = jnp.zeros((NPAD, D), jnp.float32).at[:N].set(x)
    count_kernel, scatter_kernel = _sc_kernels()
    cs, cd = count_kernel(srcm, dstm)
    cs = cs.reshape(NC, NPAD)
    cd = cd.reshape(NC, NPAD)
    xn = _norm_call(xpad, cs)
    srcm3 = srcm.reshape(NW * NBLK, IBLK, CHUNK)
    dstm3 = dstm.reshape(NW * NBLK, IBLK, CHUNK)
    p1 = scatter_kernel(xn, srcm3, dstm3)
    _, hn1, _ = _layer_call(p1, xn, cs, cd, W1, b1.reshape(1, D))
    p2 = scatter_kernel(hn1, srcm3, dstm3)
    h2, _, gsum = _layer_call(p2, hn1, cs, cd, W2, b2.reshape(1, D))
    return (gsum, h2[:N])


# speedup vs baseline: 9.1247x; 1.0003x over previous
import functools

import jax
import jax.numpy as jnp
from jax import lax
from jax.experimental import pallas as pl
from jax.experimental.pallas import tpu as pltpu
from jax.experimental.pallas import tpu_sc as plsc

N = 10000
D = 128
E = 320000

NC = 2
NS = 16
NW = NC * NS

CHUNK = 128
CPT = 80
EPT = CPT * CHUNK
EP = EPT * NW
NPAD = 10240
RPT = NPAD // NS
BN = 512

def _count_body(srcm, dstm, out_s, out_d, src_v, dst_v, ones_v, drain_v,
                acc_s, acc_d):
    c = lax.axis_index("c")
    s = lax.axis_index("s")
    w = s * NC + c

    def _zero(i, _):
        drain_v[pl.ds(i * 16, 16)] = jnp.zeros((16,), jnp.float32)
        return ()
    lax.fori_loop(0, RPT // 16, _zero, ())
    for j in range(CHUNK // 16):
        ones_v[pl.ds(j * 16, 16)] = jnp.ones((16,), jnp.float32)
    pltpu.sync_copy(drain_v, acc_s.at[pl.ds(s * RPT, RPT)])
    pltpu.sync_copy(drain_v, acc_d.at[pl.ds(s * RPT, RPT)])
    plsc.subcore_barrier()

    pltpu.sync_copy(srcm.at[pl.ds(w * CPT, CPT)], src_v)
    pltpu.sync_copy(dstm.at[pl.ds(w * CPT, CPT)], dst_v)

    def _body(g, _):
        pltpu.sync_copy(ones_v, acc_s.at[src_v.at[g]], add=True)
        pltpu.sync_copy(ones_v, acc_d.at[dst_v.at[g]], add=True)
        return ()
    lax.fori_loop(0, CPT, _body, ())

    plsc.subcore_barrier()
    pltpu.sync_copy(acc_s.at[pl.ds(s * RPT, RPT)], drain_v)
    pltpu.sync_copy(drain_v, out_s.at[pl.ds(c * NPAD + s * RPT, RPT)])
    pltpu.sync_copy(acc_d.at[pl.ds(s * RPT, RPT)], drain_v)
    pltpu.sync_copy(drain_v, out_d.at[pl.ds(c * NPAD + s * RPT, RPT)])


IBLK = 8
NBLK = CPT // IBLK
NBLK0 = 17
NBLK1 = 3


def _scatter_body(xn, srcm3, dstm3, out, sidx, didx, rows2, acc, g0, g1):
    gs = (g0, g1)
    c = lax.axis_index("c")
    s = lax.axis_index("s")
    w = s * NC + c

    def _zrow(i, _):
        for j in range(D // 16):
            rows2[0, i, pl.ds(j * 16, 16)] = jnp.zeros((16,), jnp.float32)
        return ()
    lax.fori_loop(0, CHUNK, _zrow, ())
    for k in range(RPT // CHUNK):
        pltpu.sync_copy(rows2.at[0], acc.at[pl.ds(s * RPT + k * CHUNK, CHUNK)])
    plsc.subcore_barrier()

    nblk = jnp.where(c == 0, NBLK0, NBLK1)
    blk0 = jnp.where(c == 0, s * NBLK0, NS * NBLK0 + s * NBLK1)

    def _block(bi, _):
        base = blk0 + bi
        pltpu.sync_copy(srcm3.at[base], sidx)
        pltpu.sync_copy(dstm3.at[base], didx)
        pltpu.async_copy(xn.at[sidx.at[0]], rows2.at[0], gs[0]).wait()
        for j in range(IBLK):
            if j < IBLK - 1:
                desc = pltpu.async_copy(
                    xn.at[sidx.at[j + 1]], rows2.at[(j + 1) % 2], gs[1])
            pltpu.sync_copy(rows2.at[j % 2], acc.at[didx.at[j]], add=True)
            if j < IBLK - 1:
                desc.wait()
        return ()
    lax.fori_loop(0, nblk, _block, ())

    plsc.subcore_barrier()
    pltpu.sync_copy(acc.at[pl.ds(s * RPT, RPT)], out.at[c, pl.ds(s * RPT, RPT)])


@functools.cache
def _sc_kernels():
    mesh = plsc.VectorSubcoreMesh(core_axis_name="c", subcore_axis_name="s")
    count_kernel = pl.kernel(
        _count_body,
        out_type=[
            jax.ShapeDtypeStruct((NC * NPAD,), jnp.float32),
            jax.ShapeDtypeStruct((NC * NPAD,), jnp.float32),
        ],
        mesh=mesh,
        scratch_types=[
            pltpu.VMEM((CPT, CHUNK), jnp.int32),
            pltpu.VMEM((CPT, CHUNK), jnp.int32),
            pltpu.VMEM((CHUNK,), jnp.float32),
            pltpu.VMEM((RPT,), jnp.float32),
            pltpu.VMEM_SHARED((NPAD,), jnp.float32),
            pltpu.VMEM_SHARED((NPAD,), jnp.float32),
        ],
    )
    scatter_kernel = pl.kernel(
        _scatter_body,
        out_type=jax.ShapeDtypeStruct((NC, NPAD, D), jnp.float32),
        mesh=mesh,
        scratch_types=[
            pltpu.VMEM((IBLK, CHUNK), jnp.int32),
            pltpu.VMEM((IBLK, CHUNK), jnp.int32),
            pltpu.VMEM((2, CHUNK, D), jnp.float32),
            pltpu.VMEM_SHARED((NPAD, D), jnp.float32),
            pltpu.SemaphoreType.DMA,
            pltpu.SemaphoreType.DMA,
        ],
    )
    return count_kernel, scatter_kernel


def _norm_body(x_ref, cs_ref, xn_ref):
    deg_in = cs_ref[0] + cs_ref[1] + 1.0
    xn_ref[...] = x_ref[...] * lax.rsqrt(deg_in)[:, None]


_norm_call = pl.pallas_call(
    _norm_body,
    grid=(NPAD // BN,),
    in_specs=[
        pl.BlockSpec((BN, D), lambda i: (i, 0)),
        pl.BlockSpec((NC, BN), lambda i: (0, i)),
    ],
    out_specs=pl.BlockSpec((BN, D), lambda i: (i, 0)),
    out_shape=jax.ShapeDtypeStruct((NPAD, D), jnp.float32),
)


def _layer_body(p_ref, xn_ref, cs_ref, cd_ref, w_ref, b_ref,
                h_ref, hn_ref, g_ref):
    i = pl.program_id(0)
    deg_out = cd_ref[0] + cd_ref[1] + 1.0
    deg_in = cs_ref[0] + cs_ref[1] + 1.0
    upd = (p_ref[0] + p_ref[1] + xn_ref[...]) * lax.rsqrt(deg_out)[:, None]
    h = jnp.dot(upd, w_ref[...], preferred_element_type=jnp.float32)
    h = jnp.maximum(h + b_ref[...], 0.0)
    h_ref[...] = h
    hn_ref[...] = h * lax.rsqrt(deg_in)[:, None]
    rows = jax.lax.broadcasted_iota(jnp.int32, (BN, 1), 0) + i * BN
    bsum = jnp.sum(jnp.where(rows < N, h, 0.0), axis=0, keepdims=True)

    @pl.when(i == 0)
    def _():
        g_ref[...] = bsum

    @pl.when(i > 0)
    def _():
        g_ref[...] = g_ref[...] + bsum


_layer_call = pl.pallas_call(
    _layer_body,
    grid=(NPAD // BN,),
    in_specs=[
        pl.BlockSpec((NC, BN, D), lambda i: (0, i, 0)),
        pl.BlockSpec((BN, D), lambda i: (i, 0)),
        pl.BlockSpec((NC, BN), lambda i: (0, i)),
        pl.BlockSpec((NC, BN), lambda i: (0, i)),
        pl.BlockSpec((D, D), lambda i: (0, 0)),
        pl.BlockSpec((1, D), lambda i: (0, 0)),
    ],
    out_specs=[
        pl.BlockSpec((BN, D), lambda i: (i, 0)),
        pl.BlockSpec((BN, D), lambda i: (i, 0)),
        pl.BlockSpec((1, D), lambda i: (0, 0)),
    ],
    out_shape=[
        jax.ShapeDtypeStruct((NPAD, D), jnp.float32),
        jax.ShapeDtypeStruct((NPAD, D), jnp.float32),
        jax.ShapeDtypeStruct((1, D), jnp.float32),
    ],
)


def kernel(x, edge_index, W1, b1, W2, b2):
    src = edge_index[0]
    dst = edge_index[1]
    pad = jnp.full((EP - E,), N, jnp.int32)
    srcm = jnp.concatenate([src, pad]).reshape(EP // CHUNK, CHUNK)
    dstm = jnp.concatenate([dst, pad]).reshape(EP // CHUNK, CHUNK)
    xpad = jnp.zeros((NPAD, D), jnp.float32).at[:N].set(x)
    count_kernel, scatter_kernel = _sc_kernels()
    cs, cd = count_kernel(srcm, dstm)
    cs = cs.reshape(NC, NPAD)
    cd = cd.reshape(NC, NPAD)
    xn = _norm_call(xpad, cs)
    srcm3 = srcm.reshape(NW * NBLK, IBLK, CHUNK)
    dstm3 = dstm.reshape(NW * NBLK, IBLK, CHUNK)
    p1 = scatter_kernel(xn, srcm3, dstm3)
    _, hn1, _ = _layer_call(p1, xn, cs, cd, W1, b1.reshape(1, D))
    p2 = scatter_kernel(hn1, srcm3, dstm3)
    h2, _, gsum = _layer_call(p2, hn1, cs, cd, W2, b2.reshape(1, D))
    return (gsum, h2[:N])


# speedup vs baseline: 9.1590x; 1.0038x over previous
import functools

import jax
import jax.numpy as jnp
from jax import lax
from jax.experimental import pallas as pl
from jax.experimental.pallas import tpu as pltpu
from jax.experimental.pallas import tpu_sc as plsc

N = 10000
D = 128
E = 320000

NC = 2
NS = 16
NW = NC * NS

CHUNK = 128
CPT = 80
EPT = CPT * CHUNK
EP = EPT * NW
NPAD = 10240
RPT = NPAD // NS
BN = 512

def _count_body(srcm, dstm, out_s, out_d, src_v, dst_v, ones_v, drain_v,
                acc_s, acc_d):
    c = lax.axis_index("c")
    s = lax.axis_index("s")
    w = s * NC + c

    def _zero(i, _):
        drain_v[pl.ds(i * 16, 16)] = jnp.zeros((16,), jnp.float32)
        return ()
    lax.fori_loop(0, RPT // 16, _zero, ())
    for j in range(CHUNK // 16):
        ones_v[pl.ds(j * 16, 16)] = jnp.ones((16,), jnp.float32)
    pltpu.sync_copy(drain_v, acc_s.at[pl.ds(s * RPT, RPT)])
    pltpu.sync_copy(drain_v, acc_d.at[pl.ds(s * RPT, RPT)])
    plsc.subcore_barrier()

    pltpu.sync_copy(srcm.at[pl.ds(w * CPT, CPT)], src_v)
    pltpu.sync_copy(dstm.at[pl.ds(w * CPT, CPT)], dst_v)

    def _body(g, _):
        pltpu.sync_copy(ones_v, acc_s.at[src_v.at[g]], add=True)
        pltpu.sync_copy(ones_v, acc_d.at[dst_v.at[g]], add=True)
        return ()
    lax.fori_loop(0, CPT, _body, ())

    plsc.subcore_barrier()
    pltpu.sync_copy(acc_s.at[pl.ds(s * RPT, RPT)], drain_v)
    pltpu.sync_copy(drain_v, out_s.at[pl.ds(c * NPAD + s * RPT, RPT)])
    pltpu.sync_copy(acc_d.at[pl.ds(s * RPT, RPT)], drain_v)
    pltpu.sync_copy(drain_v, out_d.at[pl.ds(c * NPAD + s * RPT, RPT)])


IBLK = 8
NBLK = CPT // IBLK
NBLK0 = 18
NBLK1 = 2


def _scatter_body(xn, srcm3, dstm3, out, sidx, didx, rows2, acc, g0, g1):
    gs = (g0, g1)
    c = lax.axis_index("c")
    s = lax.axis_index("s")
    w = s * NC + c

    def _zrow(i, _):
        for j in range(D // 16):
            rows2[0, i, pl.ds(j * 16, 16)] = jnp.zeros((16,), jnp.float32)
        return ()
    lax.fori_loop(0, CHUNK, _zrow, ())
    for k in range(RPT // CHUNK):
        pltpu.sync_copy(rows2.at[0], acc.at[pl.ds(s * RPT + k * CHUNK, CHUNK)])
    plsc.subcore_barrier()

    nblk = jnp.where(c == 0, NBLK0, NBLK1)
    blk0 = jnp.where(c == 0, s * NBLK0, NS * NBLK0 + s * NBLK1)

    def _block(bi, _):
        base = blk0 + bi
        pltpu.sync_copy(srcm3.at[base], sidx)
        pltpu.sync_copy(dstm3.at[base], didx)
        pltpu.async_copy(xn.at[sidx.at[0]], rows2.at[0], gs[0]).wait()
        for j in range(IBLK):
            if j < IBLK - 1:
                desc = pltpu.async_copy(
                    xn.at[sidx.at[j + 1]], rows2.at[(j + 1) % 2], gs[1])
            pltpu.sync_copy(rows2.at[j % 2], acc.at[didx.at[j]], add=True)
            if j < IBLK - 1:
                desc.wait()
        return ()
    lax.fori_loop(0, nblk, _block, ())

    plsc.subcore_barrier()
    pltpu.sync_copy(acc.at[pl.ds(s * RPT, RPT)], out.at[c, pl.ds(s * RPT, RPT)])


@functools.cache
def _sc_kernels():
    mesh = plsc.VectorSubcoreMesh(core_axis_name="c", subcore_axis_name="s")
    count_kernel = pl.kernel(
        _count_body,
        out_type=[
            jax.ShapeDtypeStruct((NC * NPAD,), jnp.float32),
            jax.ShapeDtypeStruct((NC * NPAD,), jnp.float32),
        ],
        mesh=mesh,
        scratch_types=[
            pltpu.VMEM((CPT, CHUNK), jnp.int32),
            pltpu.VMEM((CPT, CHUNK), jnp.int32),
            pltpu.VMEM((CHUNK,), jnp.float32),
            pltpu.VMEM((RPT,), jnp.float32),
            pltpu.VMEM_SHARED((NPAD,), jnp.float32),
            pltpu.VMEM_SHARED((NPAD,), jnp.float32),
        ],
    )
    scatter_kernel = pl.kernel(
        _scatter_body,
        out_type=jax.ShapeDtypeStruct((NC, NPAD, D), jnp.float32),
        mesh=mesh,
        scratch_types=[
            pltpu.VMEM((IBLK, CHUNK), jnp.int32),
            pltpu.VMEM((IBLK, CHUNK), jnp.int32),
            pltpu.VMEM((2, CHUNK, D), jnp.float32),
            pltpu.VMEM_SHARED((NPAD, D), jnp.float32),
            pltpu.SemaphoreType.DMA,
            pltpu.SemaphoreType.DMA,
        ],
    )
    return count_kernel, scatter_kernel


def _norm_body(x_ref, cs_ref, xn_ref):
    deg_in = cs_ref[0] + cs_ref[1] + 1.0
    xn_ref[...] = x_ref[...] * lax.rsqrt(deg_in)[:, None]


_norm_call = pl.pallas_call(
    _norm_body,
    grid=(NPAD // BN,),
    in_specs=[
        pl.BlockSpec((BN, D), lambda i: (i, 0)),
        pl.BlockSpec((NC, BN), lambda i: (0, i)),
    ],
    out_specs=pl.BlockSpec((BN, D), lambda i: (i, 0)),
    out_shape=jax.ShapeDtypeStruct((NPAD, D), jnp.float32),
)


def _layer_body(p_ref, xn_ref, cs_ref, cd_ref, w_ref, b_ref,
                h_ref, hn_ref, g_ref):
    i = pl.program_id(0)
    deg_out = cd_ref[0] + cd_ref[1] + 1.0
    deg_in = cs_ref[0] + cs_ref[1] + 1.0
    upd = (p_ref[0] + p_ref[1] + xn_ref[...]) * lax.rsqrt(deg_out)[:, None]
    h = jnp.dot(upd, w_ref[...], preferred_element_type=jnp.float32)
    h = jnp.maximum(h + b_ref[...], 0.0)
    h_ref[...] = h
    hn_ref[...] = h * lax.rsqrt(deg_in)[:, None]
    rows = jax.lax.broadcasted_iota(jnp.int32, (BN, 1), 0) + i * BN
    bsum = jnp.sum(jnp.where(rows < N, h, 0.0), axis=0, keepdims=True)

    @pl.when(i == 0)
    def _():
        g_ref[...] = bsum

    @pl.when(i > 0)
    def _():
        g_ref[...] = g_ref[...] + bsum


_layer_call = pl.pallas_call(
    _layer_body,
    grid=(NPAD // BN,),
    in_specs=[
        pl.BlockSpec((NC, BN, D), lambda i: (0, i, 0)),
        pl.BlockSpec((BN, D), lambda i: (i, 0)),
        pl.BlockSpec((NC, BN), lambda i: (0, i)),
        pl.BlockSpec((NC, BN), lambda i: (0, i)),
        pl.BlockSpec((D, D), lambda i: (0, 0)),
        pl.BlockSpec((1, D), lambda i: (0, 0)),
    ],
    out_specs=[
        pl.BlockSpec((BN, D), lambda i: (i, 0)),
        pl.BlockSpec((BN, D), lambda i: (i, 0)),
        pl.BlockSpec((1, D), lambda i: (0, 0)),
    ],
    out_shape=[
        jax.ShapeDtypeStruct((NPAD, D), jnp.float32),
        jax.ShapeDtypeStruct((NPAD, D), jnp.float32),
        jax.ShapeDtypeStruct((1, D), jnp.float32),
    ],
)


def kernel(x, edge_index, W1, b1, W2, b2):
    src = edge_index[0]
    dst = edge_index[1]
    pad = jnp.full((EP - E,), N, jnp.int32)
    srcm = jnp.concatenate([src, pad]).reshape(EP // CHUNK, CHUNK)
    dstm = jnp.concatenate([dst, pad]).reshape(EP // CHUNK, CHUNK)
    xpad = jnp.zeros((NPAD, D), jnp.float32).at[:N].set(x)
    count_kernel, scatter_kernel = _sc_kernels()
    cs, cd = count_kernel(srcm, dstm)
    cs = cs.reshape(NC, NPAD)
    cd = cd.reshape(NC, NPAD)
    xn = _norm_call(xpad, cs)
    srcm3 = srcm.reshape(NW * NBLK, IBLK, CHUNK)
    dstm3 = dstm.reshape(NW * NBLK, IBLK, CHUNK)
    p1 = scatter_kernel(xn, srcm3, dstm3)
    _, hn1, _ = _layer_call(p1, xn, cs, cd, W1, b1.reshape(1, D))
    p2 = scatter_kernel(hn1, srcm3, dstm3)
    h2, _, gsum = _layer_call(p2, hn1, cs, cd, W2, b2.reshape(1, D))
    return (gsum, h2[:N])


# speedup vs baseline: 9.1613x; 1.0003x over previous
import functools

import jax
import jax.numpy as jnp
from jax import lax
from jax.experimental import pallas as pl
from jax.experimental.pallas import tpu as pltpu
from jax.experimental.pallas import tpu_sc as plsc

N = 10000
D = 128
E = 320000

NC = 2
NS = 16
NW = NC * NS

CHUNK = 128
CPT = 80
EPT = CPT * CHUNK
EP = EPT * NW
NPAD = 10240
RPT = NPAD // NS
BN = 512

def _count_body(srcm, dstm, out_s, out_d, src_v, dst_v, ones_v, drain_v,
                acc_s, acc_d):
    c = lax.axis_index("c")
    s = lax.axis_index("s")
    w = s * NC + c

    def _zero(i, _):
        drain_v[pl.ds(i * 16, 16)] = jnp.zeros((16,), jnp.float32)
        return ()
    lax.fori_loop(0, RPT // 16, _zero, ())
    for j in range(CHUNK // 16):
        ones_v[pl.ds(j * 16, 16)] = jnp.ones((16,), jnp.float32)
    pltpu.sync_copy(drain_v, acc_s.at[pl.ds(s * RPT, RPT)])
    pltpu.sync_copy(drain_v, acc_d.at[pl.ds(s * RPT, RPT)])
    plsc.subcore_barrier()

    pltpu.sync_copy(srcm.at[pl.ds(w * CPT, CPT)], src_v)
    pltpu.sync_copy(dstm.at[pl.ds(w * CPT, CPT)], dst_v)

    def _body(g, _):
        pltpu.sync_copy(ones_v, acc_s.at[src_v.at[g]], add=True)
        pltpu.sync_copy(ones_v, acc_d.at[dst_v.at[g]], add=True)
        return ()
    lax.fori_loop(0, CPT, _body, ())

    plsc.subcore_barrier()
    pltpu.sync_copy(acc_s.at[pl.ds(s * RPT, RPT)], drain_v)
    pltpu.sync_copy(drain_v, out_s.at[pl.ds(c * NPAD + s * RPT, RPT)])
    pltpu.sync_copy(acc_d.at[pl.ds(s * RPT, RPT)], drain_v)
    pltpu.sync_copy(drain_v, out_d.at[pl.ds(c * NPAD + s * RPT, RPT)])


IBLK = 8
NBLK = CPT // IBLK
NBLK0 = 18
NBLK1 = 2


def _scatter_body(xn, srcm3, dstm3, out, sidx, didx, rows2, acc, g0, g1):
    gs = (g0, g1)
    c = lax.axis_index("c")
    s = lax.axis_index("s")

    def _zrow(i, _):
        for j in range(D // 16):
            rows2[0, i, pl.ds(j * 16, 16)] = jnp.zeros((16,), jnp.float32)
        return ()
    lax.fori_loop(0, CHUNK, _zrow, ())
    for k in range(RPT // CHUNK):
        pltpu.sync_copy(rows2.at[0], acc.at[pl.ds(s * RPT + k * CHUNK, CHUNK)])
    plsc.subcore_barrier()

    nblk = jnp.where(c == 0, NBLK0, NBLK1)
    blk0 = jnp.where(c == 0, s * NBLK0, NS * NBLK0 + s * NBLK1)

    def _block(bi, _):
        base = blk0 + bi
        pltpu.sync_copy(srcm3.at[base], sidx)
        pltpu.sync_copy(dstm3.at[base], didx)
        pltpu.async_copy(xn.at[sidx.at[0]], rows2.at[0], gs[0]).wait()
        for j in range(IBLK):
            if j < IBLK - 1:
                desc = pltpu.async_copy(
                    xn.at[sidx.at[j + 1]], rows2.at[(j + 1) % 2], gs[1])
            pltpu.sync_copy(rows2.at[j % 2], acc.at[didx.at[j]], add=True)
            if j < IBLK - 1:
                desc.wait()
        return ()
    lax.fori_loop(0, nblk, _block, ())

    plsc.subcore_barrier()
    pltpu.sync_copy(acc.at[pl.ds(s * RPT, RPT)], out.at[c, pl.ds(s * RPT, RPT)])


@functools.cache
def _sc_kernels():
    mesh = plsc.VectorSubcoreMesh(core_axis_name="c", subcore_axis_name="s")
    count_kernel = pl.kernel(
        _count_body,
        out_type=[
            jax.ShapeDtypeStruct((NC * NPAD,), jnp.float32),
            jax.ShapeDtypeStruct((NC * NPAD,), jnp.float32),
        ],
        mesh=mesh,
        scratch_types=[
            pltpu.VMEM((CPT, CHUNK), jnp.int32),
            pltpu.VMEM((CPT, CHUNK), jnp.int32),
            pltpu.VMEM((CHUNK,), jnp.float32),
            pltpu.VMEM((RPT,), jnp.float32),
            pltpu.VMEM_SHARED((NPAD,), jnp.float32),
            pltpu.VMEM_SHARED((NPAD,), jnp.float32),
        ],
    )
    scatter_kernel = pl.kernel(
        _scatter_body,
        out_type=jax.ShapeDtypeStruct((NC, NPAD, D), jnp.float32),
        mesh=mesh,
        scratch_types=[
            pltpu.VMEM((IBLK, CHUNK), jnp.int32),
            pltpu.VMEM((IBLK, CHUNK), jnp.int32),
            pltpu.VMEM((2, CHUNK, D), jnp.float32),
            pltpu.VMEM_SHARED((NPAD, D), jnp.float32),
            pltpu.SemaphoreType.DMA,
            pltpu.SemaphoreType.DMA,
        ],
    )
    return count_kernel, scatter_kernel


def _norm_body(x_ref, cs_ref, xn_ref):
    deg_in = cs_ref[0] + cs_ref[1] + 1.0
    xn_ref[...] = x_ref[...] * lax.rsqrt(deg_in)[:, None]


_norm_call = pl.pallas_call(
    _norm_body,
    grid=(NPAD // BN,),
    in_specs=[
        pl.BlockSpec((BN, D), lambda i: (i, 0)),
        pl.BlockSpec((NC, BN), lambda i: (0, i)),
    ],
    out_specs=pl.BlockSpec((BN, D), lambda i: (i, 0)),
    out_shape=jax.ShapeDtypeStruct((NPAD, D), jnp.float32),
)


def _layer_body(p_ref, xn_ref, cs_ref, cd_ref, w_ref, b_ref,
                h_ref, hn_ref, g_ref):
    i = pl.program_id(0)
    deg_out = cd_ref[0] + cd_ref[1] + 1.0
    deg_in = cs_ref[0] + cs_ref[1] + 1.0
    upd = (p_ref[0] + p_ref[1] + xn_ref[...]) * lax.rsqrt(deg_out)[:, None]
    h = jnp.dot(upd, w_ref[...], preferred_element_type=jnp.float32)
    h = jnp.maximum(h + b_ref[...], 0.0)
    h_ref[...] = h
    hn_ref[...] = h * lax.rsqrt(deg_in)[:, None]
    rows = jax.lax.broadcasted_iota(jnp.int32, (BN, 1), 0) + i * BN
    bsum = jnp.sum(jnp.where(rows < N, h, 0.0), axis=0, keepdims=True)

    @pl.when(i == 0)
    def _():
        g_ref[...] = bsum

    @pl.when(i > 0)
    def _():
        g_ref[...] = g_ref[...] + bsum


_layer_call = pl.pallas_call(
    _layer_body,
    grid=(NPAD // BN,),
    in_specs=[
        pl.BlockSpec((NC, BN, D), lambda i: (0, i, 0)),
        pl.BlockSpec((BN, D), lambda i: (i, 0)),
        pl.BlockSpec((NC, BN), lambda i: (0, i)),
        pl.BlockSpec((NC, BN), lambda i: (0, i)),
        pl.BlockSpec((D, D), lambda i: (0, 0)),
        pl.BlockSpec((1, D), lambda i: (0, 0)),
    ],
    out_specs=[
        pl.BlockSpec((BN, D), lambda i: (i, 0)),
        pl.BlockSpec((BN, D), lambda i: (i, 0)),
        pl.BlockSpec((1, D), lambda i: (0, 0)),
    ],
    out_shape=[
        jax.ShapeDtypeStruct((NPAD, D), jnp.float32),
        jax.ShapeDtypeStruct((NPAD, D), jnp.float32),
        jax.ShapeDtypeStruct((1, D), jnp.float32),
    ],
)


def kernel(x, edge_index, W1, b1, W2, b2):
    src = edge_index[0]
    dst = edge_index[1]
    pad = jnp.full((EP - E,), N, jnp.int32)
    srcm = jnp.concatenate([src, pad]).reshape(EP // CHUNK, CHUNK)
    dstm = jnp.concatenate([dst, pad]).reshape(EP // CHUNK, CHUNK)
    xpad = jnp.zeros((NPAD, D), jnp.float32).at[:N].set(x)
    count_kernel, scatter_kernel = _sc_kernels()
    cs, cd = count_kernel(srcm, dstm)
    cs = cs.reshape(NC, NPAD)
    cd = cd.reshape(NC, NPAD)
    xn = _norm_call(xpad, cs)
    srcm3 = srcm.reshape(NW * NBLK, IBLK, CHUNK)
    dstm3 = dstm.reshape(NW * NBLK, IBLK, CHUNK)
    p1 = scatter_kernel(xn, srcm3, dstm3)
    _, hn1, _ = _layer_call(p1, xn, cs, cd, W1, b1.reshape(1, D))
    p2 = scatter_kernel(hn1, srcm3, dstm3)
    h2, _, gsum = _layer_call(p2, hn1, cs, cd, W2, b2.reshape(1, D))
    return (gsum, h2[:N])
